# baseline (speedup 1.0000x reference)
import functools

import jax
import jax.numpy as jnp
from jax import lax
from jax.experimental import pallas as pl
from jax.experimental.pallas import tpu as pltpu
from jax.experimental.pallas import tpu_sc as plsc

D = 64
DP = 128
IW = 128
K = 4
CHUNK = K * IW


def _build(N, V):
    info = plsc.get_sparse_core_info()
    NC, NS = info.num_cores, info.num_subcores
    NW = NC * NS
    assert N % (NW * CHUNK) == 0
    b_per_w = N // NW
    n_chunks = b_per_w // CHUNK
    assert n_chunks % 4 == 0
    G = n_chunks // 4

    mesh = plsc.VectorSubcoreMesh(core_axis_name="c", subcore_axis_name="s")

    @functools.partial(
        pl.kernel,
        mesh=mesh,
        out_type=jax.ShapeDtypeStruct((N, DP), jnp.float32),
        compiler_params=pltpu.CompilerParams(use_tc_tiling_on_sc=False),
        scratch_types=[
            pltpu.VMEM((4, K, IW), jnp.int32),
            pltpu.VMEM((2, CHUNK, D), jnp.float32),
            pltpu.VMEM_SHARED((V, D), jnp.float32),
            pltpu.SemaphoreType.DMA,
            pltpu.SemaphoreType.DMA,
            pltpu.SemaphoreType.DMA,
            pltpu.SemaphoreType.DMA,
            pltpu.SemaphoreType.DMA,
            pltpu.SemaphoreType.DMA,
            pltpu.SemaphoreType.DMA,
            pltpu.SemaphoreType.DMA,
        ],
    )
    def k(table_hbm, idx_hbm, out_hbm, idx_v, rows_v, table_sh,
          gat_sem0, gat_sem1, idx_sem0, idx_sem1, idx_sem2, idx_sem3,
          out_sem0, out_sem1):
        wid = lax.axis_index("s") * NC + lax.axis_index("c")
        base = wid * b_per_w
        rbase = base // IW

        @pl.when(lax.axis_index("s") == 0)
        def _stage():
            pltpu.sync_copy(table_hbm, table_sh)
        plsc.subcore_barrier()

        gat_sems = (gat_sem0, gat_sem1)
        idx_sems = (idx_sem0, idx_sem1, idx_sem2, idx_sem3)
        out_sems = (out_sem0, out_sem1)

        def idx_copy(i, s):
            return pltpu.make_async_copy(
                idx_hbm.at[pl.ds(rbase + i * K, K)], idx_v.at[s], idx_sems[s]
            )

        def out_copy(i, b):
            return pltpu.make_async_copy(
                rows_v.at[b],
                out_hbm.at[pl.ds(base + i * CHUNK, CHUNK), pl.ds(0, D)],
                out_sems[b],
            )

        def fire_gathers(s, b):
            for j in range(K):
                pltpu.async_copy(
                    table_sh.at[idx_v.at[s, j]],
                    rows_v.at[b, pl.ds(j * IW, IW)],
                    gat_sems[b],
                )

        def drain_gathers(b):
            for j in range(K):
                pltpu.make_async_copy(
                    table_sh.at[idx_v.at[0, 0]],
                    rows_v.at[b, pl.ds(j * IW, IW)],
                    gat_sems[b],
                ).wait()

        idx_copy(0, 0).start()
        idx_copy(1, 1).start()
        idx_copy(2, 2).start()

        def step(i, p):
            b = p % 2
            idx_copy(i, p).wait()

            if not (isinstance(i, int) and i < 2):
                out_copy(i, b).wait()
            fire_gathers(p, b)

            if not (isinstance(i, int) and i < 1):
                drain_gathers(1 - b)
                out_copy(i - 1, 1 - b).start()
                if isinstance(i, int):
                    if i + 2 < n_chunks:
                        idx_copy(i + 2, (p + 2) % 4).start()
                else:
                    @pl.when(i + 2 < n_chunks)
                    def _prefetch():
                        idx_copy(i + 2, (p + 2) % 4).start()

        def body(g, _):
            for p in range(4):
                step(4 * g + p, p)
            return 0

        for p in range(4):
            step(p, p)
        lax.fori_loop(1, G, body, 0)

        last = n_chunks - 1
        drain_gathers(last % 2)
        out_copy(last, last % 2).start()
        out_copy(0, 0).wait()
        out_copy(0, 1).wait()

    return k


def kernel(positions, table):
    B, S = positions.shape
    V, d = table.shape
    N = B * S
    idx = positions.reshape(N // IW, IW).astype(jnp.int32)
    out = _build(N, V)(table, idx)
    return out[:, :d].reshape(B, S, d)
